# baseline (speedup 1.0000x reference)
import functools

import jax
import jax.numpy as jnp
from jax import lax
from jax.experimental import pallas as pl
from jax.experimental.pallas import tpu as pltpu
from jax.experimental.pallas import tpu_sc as plsc

NB_CLASSES = 1000000
NB_RELS = 1000
D = 32
B = 16384
GAMMA = 10.0

NC, NS, L = 2, 16, 16
NW = NC * NS
BPW = B // NW
CHUNK = 64
NCHUNK = BPW // CHUNK
GPC = CHUNK // L

RB = 32768
RGRID = (NB_CLASSES + RB - 1) // RB
RB8 = RB // 8
RROWS = RGRID * RB8
RB_LOG = RB.bit_length() - 1
E_LOG = RB_LOG - 3
E_MASK = RB8 - 1


def _repack_body(ce_ref, co_ref, cep_ref, cop_ref):
    def pack(ref):
        x = ref[...]
        xs = jnp.concatenate(
            [x[:, a * RB8:(a + 1) * RB8] for a in range(8)], axis=0)
        t = jnp.transpose(xs)
        v = lax.bitcast_convert_type(t, jnp.int32)
        r = v + jnp.int32(0x7FFF) + \
            jnp.bitwise_and(lax.shift_right_logical(v, 16), 1)
        hi = jnp.bitwise_and(r[:, :4 * D], jnp.int32(-65536))
        lo = lax.shift_right_logical(r[:, 4 * D:], 16)
        return jnp.bitwise_or(hi, lo)

    cep_ref[...] = pack(ce_ref)
    cop_ref[...] = pack(co_ref)


def _repack(ceT, coT):
    return pl.pallas_call(
        _repack_body,
        grid=(RGRID,),
        in_specs=[pl.BlockSpec((D, RB), lambda i: (0, i)),
                  pl.BlockSpec((D, RB), lambda i: (0, i))],
        out_specs=[pl.BlockSpec((RB8, 4 * D), lambda i: (i, 0)),
                   pl.BlockSpec((RB8, 4 * D), lambda i: (i, 0))],
        out_shape=[jax.ShapeDtypeStruct((RROWS, 4 * D), jnp.int32),
                   jax.ShapeDtypeStruct((RROWS, 4 * D), jnp.int32)],
        compiler_params=pltpu.CompilerParams(
            fuse_transposed_lhs_in_matmul=True),
    )(ceT, coT)


def _sqrt16(x):
    xs = jnp.maximum(x, jnp.float32(1e-30))
    i = lax.bitcast_convert_type(xs, jnp.int32)
    i = jnp.int32(0x5F3759DF) - lax.shift_right_logical(i, 1)
    y = lax.bitcast_convert_type(i, jnp.float32)
    half = jnp.float32(0.5) * xs
    for _ in range(3):
        y = y * (jnp.float32(1.5) - half * y * y)
    return x * y


def _body(heads_hbm, rels_hbm, tails_hbm, cep_hbm, cop_hbm, rel_hbm, out_hbm,
          hv, rv, tv, hrow, trow, hc0, ho0, tc0, r0, hc1, ho1, tc1, r1,
          hc2, ho2, tc2, r2, out_v, sem0, sem1, sem2):
    cid = lax.axis_index("c")
    sid = lax.axis_index("s")
    wid = sid * NC + cid
    base = wid * BPW

    pltpu.sync_copy(heads_hbm.at[pl.ds(base, BPW)], hv)
    pltpu.sync_copy(rels_hbm.at[pl.ds(base, BPW)], rv)
    pltpu.sync_copy(tails_hbm.at[pl.ds(base, BPW)], tv)

    def rowify(k, carry):
        sl = pl.ds(k * L, L)
        h = hv[sl]
        t = tv[sl]
        hrow[sl] = lax.shift_left(lax.shift_right_logical(h, RB_LOG), E_LOG) + \
            jnp.bitwise_and(h, E_MASK)
        trow[sl] = lax.shift_left(lax.shift_right_logical(t, RB_LOG), E_LOG) + \
            jnp.bitwise_and(t, E_MASK)
        return carry
    lax.fori_loop(0, BPW // L, rowify, 0)

    lanes = lax.iota(jnp.int32, L)
    bufsets = ((hc0, ho0, tc0, r0, sem0), (hc1, ho1, tc1, r1, sem1),
               (hc2, ho2, tc2, r2, sem2))

    def fire(c, bs):
        hc_buf, ho_buf, tc_buf, r_buf, sem = bs
        isl = pl.ds(c * CHUNK, CHUNK)
        pltpu.async_copy(cep_hbm.at[hrow.at[isl]], hc_buf, sem)
        pltpu.async_copy(cop_hbm.at[hrow.at[isl]], ho_buf, sem)
        pltpu.async_copy(cep_hbm.at[trow.at[isl]], tc_buf, sem)
        pltpu.async_copy(rel_hbm.at[rv.at[isl]], r_buf, sem)

    def drain(c, bs):
        hc_buf, ho_buf, tc_buf, r_buf, sem = bs
        isl = pl.ds(c * CHUNK, CHUNK)
        pltpu.make_async_copy(cep_hbm.at[hrow.at[isl]], hc_buf, sem).wait()
        pltpu.make_async_copy(cop_hbm.at[hrow.at[isl]], ho_buf, sem).wait()
        pltpu.make_async_copy(cep_hbm.at[trow.at[isl]], tc_buf, sem).wait()
        pltpu.make_async_copy(rel_hbm.at[rv.at[isl]], r_buf, sem).wait()

    def compute(j, bs):
        hc_buf, ho_buf, tc_buf, r_buf, _ = bs

        def group(g, gcarry):
            rows = g * L + lanes
            sl = pl.ds(j * CHUNK + g * L, L)
            h = hv[sl]
            t = tv[sl]
            hs = jnp.bitwise_and(lax.shift_right_logical(h, E_LOG), 7)
            ts = jnp.bitwise_and(lax.shift_right_logical(t, E_LOG), 7)
            hq = lax.shift_left(jnp.bitwise_and(hs, 3), 5)
            tq = lax.shift_left(jnp.bitwise_and(ts, 3), 5)
            h_hi = hs < 4
            t_hi = ts < 4

            def widen(xi, hi_mask):
                bits = jnp.where(hi_mask, jnp.bitwise_and(xi, jnp.int32(-65536)),
                                 lax.shift_left(xi, 16))
                return lax.bitcast_convert_type(bits, jnp.float32)

            acc_o = jnp.zeros((L,), jnp.float32)
            acc_i = jnp.zeros((L,), jnp.float32)
            for d in range(D):
                cc = widen(plsc.load_gather(hc_buf, [rows, hq + d]), h_hi)
                oo = widen(plsc.load_gather(ho_buf, [rows, hq + d]), h_hi)
                aa = widen(plsc.load_gather(tc_buf, [rows, tq + d]), t_hi)
                rt = plsc.load_gather(r_buf, [rows, jnp.full((L,), d, jnp.int32)])
                rf = plsc.load_gather(r_buf, [rows, jnp.full((L,), D + d, jnp.int32)])
                rs = plsc.load_gather(r_buf, [rows, jnp.full((L,), 2 * D + d, jnp.int32)])
                rb = plsc.load_gather(r_buf, [rows, jnp.full((L,), 3 * D + d, jnp.int32)])
                cc = cc * rf + rt
                off = jnp.abs(oo) * jnp.abs(rs) + jnp.abs(rb)
                delta = jnp.abs(cc - aa)
                dout = jnp.maximum(delta - off, jnp.float32(0.0))
                din = jnp.minimum(delta, off)
                acc_o = acc_o + dout * dout
                acc_i = acc_i + din * din
            dist = _sqrt16(acc_o) + jnp.float32(0.5) * _sqrt16(acc_i)
            out_v[pl.ds(j * CHUNK + g * L, L)] = jnp.float32(GAMMA) - dist
            return gcarry

        lax.fori_loop(0, GPC, group, 0)

    fire(0, bufsets[0])
    fire(1, bufsets[1])
    fire(2, bufsets[2])

    def chunk_step(c, carry):
        for s in range(3):
            @pl.when(c % 3 == s)
            def _(s=s):
                bs = bufsets[s]
                drain(c, bs)
                compute(c, bs)

                @pl.when(c + 3 < NCHUNK)
                def __():
                    fire(c + 3, bs)

        return carry

    lax.fori_loop(0, NCHUNK, chunk_step, 0)

    pltpu.sync_copy(out_v, out_hbm.at[pl.ds(base, BPW)])


@jax.jit
def _run(heads, rels, tails, cep, cop, rel_all):
    mesh = plsc.VectorSubcoreMesh(core_axis_name="c", subcore_axis_name="s")
    k = functools.partial(
        pl.kernel,
        out_type=jax.ShapeDtypeStruct((B,), jnp.float32),
        mesh=mesh,
        compiler_params=pltpu.CompilerParams(
            needs_layout_passes=False, use_tc_tiling_on_sc=True),
        scratch_types=[
            pltpu.VMEM((BPW,), jnp.int32),
            pltpu.VMEM((BPW,), jnp.int32),
            pltpu.VMEM((BPW,), jnp.int32),
            pltpu.VMEM((BPW,), jnp.int32),
            pltpu.VMEM((BPW,), jnp.int32),
            pltpu.VMEM((CHUNK, 4 * D), jnp.int32),
            pltpu.VMEM((CHUNK, 4 * D), jnp.int32),
            pltpu.VMEM((CHUNK, 4 * D), jnp.int32),
            pltpu.VMEM((CHUNK, 4 * D), jnp.float32),
            pltpu.VMEM((CHUNK, 4 * D), jnp.int32),
            pltpu.VMEM((CHUNK, 4 * D), jnp.int32),
            pltpu.VMEM((CHUNK, 4 * D), jnp.int32),
            pltpu.VMEM((CHUNK, 4 * D), jnp.float32),
            pltpu.VMEM((CHUNK, 4 * D), jnp.int32),
            pltpu.VMEM((CHUNK, 4 * D), jnp.int32),
            pltpu.VMEM((CHUNK, 4 * D), jnp.int32),
            pltpu.VMEM((CHUNK, 4 * D), jnp.float32),
            pltpu.VMEM((BPW,), jnp.float32),
            pltpu.SemaphoreType.DMA,
            pltpu.SemaphoreType.DMA,
            pltpu.SemaphoreType.DMA,
        ],
    )(_body)
    return k(heads, rels, tails, cep, cop, rel_all)


def kernel(heads, rels, tails, class_embed, class_offset, rel_embed,
           rel_factor, scale_embed, scale_bias):
    cep, cop = _repack(class_embed.T, class_offset.T)
    rel_all = jnp.concatenate(
        [rel_embed, rel_factor, scale_embed, scale_bias], axis=1)
    return _run(heads.astype(jnp.int32), rels.astype(jnp.int32),
                tails.astype(jnp.int32), cep, cop, rel_all)


# speedup vs baseline: 1.0233x; 1.0233x over previous
import functools

import jax
import jax.numpy as jnp
from jax import lax
from jax.experimental import pallas as pl
from jax.experimental.pallas import tpu as pltpu
from jax.experimental.pallas import tpu_sc as plsc

NB_CLASSES = 1000000
NB_RELS = 1000
D = 32
B = 16384
GAMMA = 10.0

NC, NS, L = 2, 16, 16
NW = NC * NS
BPW = B // NW
CHUNK = 64
NCHUNK = BPW // CHUNK
GPC = CHUNK // L

RB = 32768
RGRID = (NB_CLASSES + RB - 1) // RB
RB4 = RB // 4
RROWS = RGRID * RB4
RB_LOG = RB.bit_length() - 1
S_LOG = RB_LOG - 2
S_MASK = RB4 - 1


def _round_bf16(v):
    return v + jnp.int32(0x7FFF) + \
        jnp.bitwise_and(lax.shift_right_logical(v, 16), 1)


def _repack_body(ce_ref, co_ref, fused_ref):
    hi_parts, lo_parts = [], []
    for s in range(4):
        sl = slice(s * RB4, (s + 1) * RB4)
        hi_parts += [ce_ref[0:16, sl], co_ref[0:16, sl]]
        lo_parts += [ce_ref[16:32, sl], co_ref[16:32, sl]]
    t_hi = jnp.transpose(jnp.concatenate(hi_parts, axis=0))
    t_lo = jnp.transpose(jnp.concatenate(lo_parts, axis=0))
    hi = jnp.bitwise_and(_round_bf16(lax.bitcast_convert_type(t_hi, jnp.int32)),
                         jnp.int32(-65536))
    lo = lax.shift_right_logical(
        _round_bf16(lax.bitcast_convert_type(t_lo, jnp.int32)), 16)
    fused_ref[...] = jnp.bitwise_or(hi, lo)


def _repack(ceT, coT):
    return pl.pallas_call(
        _repack_body,
        grid=(RGRID,),
        in_specs=[pl.BlockSpec((D, RB), lambda i: (0, i)),
                  pl.BlockSpec((D, RB), lambda i: (0, i))],
        out_specs=pl.BlockSpec((RB4, 4 * D), lambda i: (i, 0)),
        out_shape=jax.ShapeDtypeStruct((RROWS, 4 * D), jnp.int32),
        compiler_params=pltpu.CompilerParams(
            fuse_transposed_lhs_in_matmul=True),
    )(ceT, coT)


def _sqrt16(x):
    xs = jnp.maximum(x, jnp.float32(1e-30))
    i = lax.bitcast_convert_type(xs, jnp.int32)
    i = jnp.int32(0x5F3759DF) - lax.shift_right_logical(i, 1)
    y = lax.bitcast_convert_type(i, jnp.float32)
    half = jnp.float32(0.5) * xs
    for _ in range(3):
        y = y * (jnp.float32(1.5) - half * y * y)
    return x * y


def _body(heads_hbm, rels_hbm, tails_hbm, fused_hbm, rel_hbm, out_hbm,
          hv, rv, tv, hrow, trow, h0, t0, r0, h1, t1, r1,
          out_v, sem0, sem1):
    cid = lax.axis_index("c")
    sid = lax.axis_index("s")
    wid = sid * NC + cid
    base = wid * BPW

    pltpu.sync_copy(heads_hbm.at[pl.ds(base, BPW)], hv)
    pltpu.sync_copy(rels_hbm.at[pl.ds(base, BPW)], rv)
    pltpu.sync_copy(tails_hbm.at[pl.ds(base, BPW)], tv)

    def rowify(k, carry):
        sl = pl.ds(k * L, L)
        h = hv[sl]
        t = tv[sl]
        hrow[sl] = lax.shift_left(lax.shift_right_logical(h, RB_LOG), S_LOG) + \
            jnp.bitwise_and(h, S_MASK)
        trow[sl] = lax.shift_left(lax.shift_right_logical(t, RB_LOG), S_LOG) + \
            jnp.bitwise_and(t, S_MASK)
        return carry
    lax.fori_loop(0, BPW // L, rowify, 0)

    lanes = lax.iota(jnp.int32, L)
    bufsets = ((h0, t0, r0, sem0), (h1, t1, r1, sem1))

    def fire(c, bs):
        h_buf, t_buf, r_buf, sem = bs
        isl = pl.ds(c * CHUNK, CHUNK)
        pltpu.async_copy(fused_hbm.at[hrow.at[isl]], h_buf, sem)
        pltpu.async_copy(fused_hbm.at[trow.at[isl]], t_buf, sem)
        pltpu.async_copy(rel_hbm.at[rv.at[isl]], r_buf, sem)

    def drain(c, bs):
        h_buf, t_buf, r_buf, sem = bs
        isl = pl.ds(c * CHUNK, CHUNK)
        pltpu.make_async_copy(fused_hbm.at[hrow.at[isl]], h_buf, sem).wait()
        pltpu.make_async_copy(fused_hbm.at[trow.at[isl]], t_buf, sem).wait()
        pltpu.make_async_copy(rel_hbm.at[rv.at[isl]], r_buf, sem).wait()

    def compute(j, bs):
        h_buf, t_buf, r_buf, _ = bs

        def group(g, gcarry):
            rows = g * L + lanes
            sl = pl.ds(j * CHUNK + g * L, L)
            h = hv[sl]
            t = tv[sl]
            hq = lax.shift_left(
                jnp.bitwise_and(lax.shift_right_logical(h, S_LOG), 3), 5)
            tq = lax.shift_left(
                jnp.bitwise_and(lax.shift_right_logical(t, S_LOG), 3), 5)

            def widen(xi, is_hi):
                if is_hi:
                    bits = jnp.bitwise_and(xi, jnp.int32(-65536))
                else:
                    bits = lax.shift_left(xi, 16)
                return lax.bitcast_convert_type(bits, jnp.float32)

            acc_o = jnp.zeros((L,), jnp.float32)
            acc_i = jnp.zeros((L,), jnp.float32)
            for d in range(D):
                is_hi = d < 16
                k = d & 15
                cc = widen(plsc.load_gather(h_buf, [rows, hq + k]), is_hi)
                oo = widen(plsc.load_gather(h_buf, [rows, hq + 16 + k]), is_hi)
                aa = widen(plsc.load_gather(t_buf, [rows, tq + k]), is_hi)
                rt = plsc.load_gather(r_buf, [rows, jnp.full((L,), d, jnp.int32)])
                rf = plsc.load_gather(r_buf, [rows, jnp.full((L,), D + d, jnp.int32)])
                rs = plsc.load_gather(r_buf, [rows, jnp.full((L,), 2 * D + d, jnp.int32)])
                rb = plsc.load_gather(r_buf, [rows, jnp.full((L,), 3 * D + d, jnp.int32)])
                cc = cc * rf + rt
                off = jnp.abs(oo) * jnp.abs(rs) + jnp.abs(rb)
                delta = jnp.abs(cc - aa)
                dout = jnp.maximum(delta - off, jnp.float32(0.0))
                din = jnp.minimum(delta, off)
                acc_o = acc_o + dout * dout
                acc_i = acc_i + din * din
            dist = _sqrt16(acc_o) + jnp.float32(0.5) * _sqrt16(acc_i)
            out_v[pl.ds(j * CHUNK + g * L, L)] = jnp.float32(GAMMA) - dist
            return gcarry

        lax.fori_loop(0, GPC, group, 0)

    fire(0, bufsets[0])

    def pair(j2, carry):
        c0 = j2 * 2
        fire(c0 + 1, bufsets[1])
        drain(c0, bufsets[0])
        compute(c0, bufsets[0])

        @pl.when(j2 < NCHUNK // 2 - 1)
        def _():
            fire(c0 + 2, bufsets[0])

        drain(c0 + 1, bufsets[1])
        compute(c0 + 1, bufsets[1])
        return carry

    lax.fori_loop(0, NCHUNK // 2, pair, 0)

    pltpu.sync_copy(out_v, out_hbm.at[pl.ds(base, BPW)])


@jax.jit
def _run(heads, rels, tails, fused, rel_all):
    mesh = plsc.VectorSubcoreMesh(core_axis_name="c", subcore_axis_name="s")
    k = functools.partial(
        pl.kernel,
        out_type=jax.ShapeDtypeStruct((B,), jnp.float32),
        mesh=mesh,
        compiler_params=pltpu.CompilerParams(
            needs_layout_passes=False, use_tc_tiling_on_sc=True),
        scratch_types=[
            pltpu.VMEM((BPW,), jnp.int32),
            pltpu.VMEM((BPW,), jnp.int32),
            pltpu.VMEM((BPW,), jnp.int32),
            pltpu.VMEM((BPW,), jnp.int32),
            pltpu.VMEM((BPW,), jnp.int32),
            pltpu.VMEM((CHUNK, 4 * D), jnp.int32),
            pltpu.VMEM((CHUNK, 4 * D), jnp.int32),
            pltpu.VMEM((CHUNK, 4 * D), jnp.float32),
            pltpu.VMEM((CHUNK, 4 * D), jnp.int32),
            pltpu.VMEM((CHUNK, 4 * D), jnp.int32),
            pltpu.VMEM((CHUNK, 4 * D), jnp.float32),
            pltpu.VMEM((BPW,), jnp.float32),
            pltpu.SemaphoreType.DMA,
            pltpu.SemaphoreType.DMA,
        ],
    )(_body)
    return k(heads, rels, tails, fused, rel_all)


def kernel(heads, rels, tails, class_embed, class_offset, rel_embed,
           rel_factor, scale_embed, scale_bias):
    fused = _repack(class_embed.T, class_offset.T)
    rel_all = jnp.concatenate(
        [rel_embed, rel_factor, scale_embed, scale_bias], axis=1)
    return _run(heads.astype(jnp.int32), rels.astype(jnp.int32),
                tails.astype(jnp.int32), fused, rel_all)
